# baseline (speedup 1.0000x reference)
import functools

import jax
import jax.numpy as jnp
from jax import lax
from jax.experimental import pallas as pl
from jax.experimental.pallas import tpu as pltpu
from jax.experimental.pallas import tpu_sc as plsc

_B = 16384
_D = 64
_SHARED = 16


@functools.cache
def _build():
    try:
        info = plsc.get_sparse_core_info()
        nc, ns = info.num_cores, info.num_subcores
    except Exception:
        nc, ns = 2, 16
    nw = nc * ns
    bpw = _B // nw
    mesh = plsc.VectorSubcoreMesh(core_axis_name="c", subcore_axis_name="s")

    @functools.partial(
        pl.kernel,
        mesh=mesh,
        out_type=jax.ShapeDtypeStruct((_B, _D), jnp.float32),
        compiler_params=pltpu.CompilerParams(use_tc_tiling_on_sc=False),
        scratch_types=[
            pltpu.VMEM((bpw,), jnp.int32),
            pltpu.VMEM((bpw, _D), jnp.float32),
            pltpu.VMEM((_SHARED,), jnp.float32),
            pltpu.SemaphoreType.DMA,
        ],
    )
    def gather_kernel(x_hbm, table_hbm, shared_hbm, out_hbm,
                      idx_v, rows_v, shared_v, sem):
        wid = lax.axis_index("s") * nc + lax.axis_index("c")
        base = wid * bpw
        pltpu.sync_copy(x_hbm.at[pl.ds(base, bpw)], idx_v)
        cp = pltpu.async_copy(table_hbm.at[idx_v], rows_v, sem)
        pltpu.sync_copy(shared_hbm, shared_v)
        cp.wait()
        svec = shared_v[...]

        def row(i, carry):
            rows_v[i, pl.ds(0, _SHARED)] = svec
            return carry

        lax.fori_loop(0, bpw, row, 0)
        pltpu.sync_copy(rows_v, out_hbm.at[pl.ds(base, bpw)])

    return gather_kernel


def kernel(X, table, shared_embed):
    return _build()(X, table, shared_embed.reshape(_SHARED))


# speedup vs baseline: 1.7227x; 1.7227x over previous
import functools

import jax
import jax.numpy as jnp
from jax import lax
from jax.experimental import pallas as pl
from jax.experimental.pallas import tpu as pltpu
from jax.experimental.pallas import tpu_sc as plsc

_B = 16384
_D = 64
_SHARED = 16
_REST = _D - _SHARED


@functools.cache
def _build():
    try:
        info = plsc.get_sparse_core_info()
        nc, ns = info.num_cores, info.num_subcores
    except Exception:
        nc, ns = 2, 16
    nw = nc * ns
    bpw = _B // nw
    mesh = plsc.VectorSubcoreMesh(core_axis_name="c", subcore_axis_name="s")

    @functools.partial(
        pl.kernel,
        mesh=mesh,
        out_type=jax.ShapeDtypeStruct((_B, _D), jnp.float32),
        scratch_types=[
            pltpu.VMEM((bpw,), jnp.int32),
            pltpu.VMEM((bpw, _D), jnp.float32),
            pltpu.VMEM((_SHARED,), jnp.float32),
            pltpu.SemaphoreType.DMA,
        ],
    )
    def gather_kernel(x_hbm, table_hbm, shared_hbm, out_hbm,
                      idx_v, rows_v, shared_v, sem):
        wid = lax.axis_index("s") * nc + lax.axis_index("c")
        base = wid * bpw
        pltpu.sync_copy(x_hbm.at[pl.ds(base, bpw)], idx_v)
        pltpu.sync_copy(shared_hbm, shared_v)

        def fire(g, carry):
            b0 = g * 16
            vi = idx_v[pl.ds(b0, 16)]
            for j in range(16):
                r = vi[j]
                pltpu.async_copy(
                    table_hbm.at[pl.ds(r, 1), pl.ds(_SHARED, _REST)],
                    rows_v.at[pl.ds(b0 + j, 1), pl.ds(_SHARED, _REST)],
                    sem,
                )
            return carry

        lax.fori_loop(0, bpw // 16, fire, 0)

        svec = shared_v[...]

        def prefill(i, carry):
            rows_v[i, pl.ds(0, _SHARED)] = svec
            return carry

        lax.fori_loop(0, bpw, prefill, 0)

        def drain(i, carry):
            pltpu.make_async_copy(
                table_hbm.at[pl.ds(0, 1), pl.ds(_SHARED, _REST)],
                rows_v.at[pl.ds(i, 1), pl.ds(_SHARED, _REST)],
                sem,
            ).wait()
            return carry

        lax.fori_loop(0, bpw, drain, 0)
        pltpu.sync_copy(rows_v, out_hbm.at[pl.ds(base, bpw)])

    return gather_kernel


def kernel(X, table, shared_embed):
    return _build()(X, table, shared_embed.reshape(_SHARED))


# speedup vs baseline: 1.7273x; 1.0026x over previous
import functools

import jax
import jax.numpy as jnp
from jax import lax
from jax.experimental import pallas as pl
from jax.experimental.pallas import tpu as pltpu
from jax.experimental.pallas import tpu_sc as plsc

_B = 16384
_D = 64
_SHARED = 16
_REST = _D - _SHARED
_NSEM = 8


@functools.cache
def _build():
    try:
        info = plsc.get_sparse_core_info()
        nc, ns = info.num_cores, info.num_subcores
    except Exception:
        nc, ns = 2, 16
    nw = nc * ns
    bpw = _B // nw
    mesh = plsc.VectorSubcoreMesh(core_axis_name="c", subcore_axis_name="s")

    @functools.partial(
        pl.kernel,
        mesh=mesh,
        out_type=jax.ShapeDtypeStruct((_B, _D), jnp.float32),
        scratch_types=[
            pltpu.VMEM((bpw,), jnp.int32),
            pltpu.VMEM((bpw, _D), jnp.float32),
            pltpu.VMEM((_SHARED,), jnp.float32),
        ]
        + [pltpu.SemaphoreType.DMA] * _NSEM,
    )
    def gather_kernel(x_hbm, table_hbm, shared_hbm, out_hbm,
                      idx_v, rows_v, shared_v, *sems):
        wid = lax.axis_index("s") * nc + lax.axis_index("c")
        base = wid * bpw
        pltpu.sync_copy(x_hbm.at[pl.ds(base, bpw)], idx_v)
        pltpu.sync_copy(shared_hbm, shared_v)

        def fire(g, carry):
            b0 = g * 16
            vi = idx_v[pl.ds(b0, 16)]
            for j in range(16):
                r = vi[j]
                pltpu.async_copy(
                    table_hbm.at[pl.ds(r, 1), pl.ds(_SHARED, _REST)],
                    rows_v.at[pl.ds(b0 + j, 1), pl.ds(_SHARED, _REST)],
                    sems[j % _NSEM],
                )
            return carry

        lax.fori_loop(0, bpw // 16, fire, 0)

        svec = shared_v[...]

        def prefill(i, carry):
            rows_v[i, pl.ds(0, _SHARED)] = svec
            return carry

        lax.fori_loop(0, bpw, prefill, 0)

        def drain(g, carry):
            for j in range(16):
                pltpu.make_async_copy(
                    table_hbm.at[pl.ds(0, 1), pl.ds(_SHARED, _REST)],
                    rows_v.at[pl.ds(g * 16 + j, 1), pl.ds(_SHARED, _REST)],
                    sems[j % _NSEM],
                ).wait()
            return carry

        lax.fori_loop(0, bpw // 16, drain, 0)
        pltpu.sync_copy(rows_v, out_hbm.at[pl.ds(base, bpw)])

    return gather_kernel


def kernel(X, table, shared_embed):
    return _build()(X, table, shared_embed.reshape(_SHARED))
